# Initial kernel scaffold; baseline (speedup 1.0000x reference)
import jax
import jax.numpy as jnp
from jax.experimental import pallas as pl


def kernel(x, types, table_0, table_1, table_2, table_3):
    raise NotImplementedError("write your pallas kernel here")


# speedup vs baseline: 9.5374x; 9.5374x over previous
import functools

import jax
import jax.numpy as jnp
from jax import lax
from jax.experimental import pallas as pl
from jax.experimental.pallas import tpu as pltpu
from jax.experimental.pallas import tpu_sc as plsc

NUM_TYPES = 4
VOCAB = 100000
EMBED = 32
N = 425984

NC = 2
NS = 16
NW = NC * NS
B_PER_W = N // NW
CHUNK = 1024
N_CHUNKS = B_PER_W // CHUNK

_mesh = plsc.VectorSubcoreMesh(core_axis_name="c", subcore_axis_name="s")


@functools.partial(
    pl.kernel,
    mesh=_mesh,
    out_type=jax.ShapeDtypeStruct((N, EMBED), jnp.float32),
    compiler_params=pltpu.CompilerParams(use_tc_tiling_on_sc=False),
    scratch_types=[
        pltpu.VMEM((CHUNK,), jnp.int32),
        pltpu.VMEM((CHUNK,), jnp.int32),
        pltpu.VMEM((CHUNK,), jnp.int32),
        pltpu.VMEM((CHUNK, EMBED), jnp.float32),
        pltpu.SemaphoreType.DMA,
    ],
)
def _hetero_gather(x_hbm, types_hbm, table_hbm, out_hbm,
                   x_v, t_v, idx_v, rows_v, sem):
    wid = lax.axis_index("s") * NC + lax.axis_index("c")
    base_w = wid * B_PER_W

    def chunk_body(c, carry):
        base = base_w + c * CHUNK
        pltpu.sync_copy(x_hbm.at[pl.ds(base, CHUNK)], x_v)
        pltpu.sync_copy(types_hbm.at[pl.ds(base, CHUNK)], t_v)

        def idx_body(i, carry2):
            off = i * 16
            idx_v[pl.ds(off, 16)] = x_v[pl.ds(off, 16)] + t_v[pl.ds(off, 16)] * VOCAB
            return carry2

        lax.fori_loop(0, CHUNK // 16, idx_body, 0)
        pltpu.async_copy(table_hbm.at[idx_v], rows_v, sem).wait()
        pltpu.sync_copy(rows_v, out_hbm.at[pl.ds(base, CHUNK)])
        return carry

    lax.fori_loop(0, N_CHUNKS, chunk_body, 0)


def kernel(x, types, table_0, table_1, table_2, table_3):
    table = jnp.concatenate([table_0, table_1, table_2, table_3], axis=0)
    return _hetero_gather(x.astype(jnp.int32), types.astype(jnp.int32), table)
